# baseline (speedup 1.0000x reference)
import jax
import jax.numpy as jnp
from jax.experimental import pallas as pl
from jax.experimental.pallas import tpu as pltpu


def _masked_copy_body(x_ref, mid_ref, o_ref):
    b = pl.program_id(0)
    mid = mid_ref[b]
    hw = x_ref.shape[-1]
    pos = jax.lax.broadcasted_iota(jnp.int32, (1, 1, hw), 2)
    o_ref[...] = jnp.where(pos == mid, 0.0, x_ref[...])


def kernel(fea, mask_id):
    b, c, h, w = fea.shape
    hw = h * w
    x = fea.reshape(b, c, hw)
    out = pl.pallas_call(
        _masked_copy_body,
        grid=(b,),
        in_specs=[
            pl.BlockSpec((1, c, hw), lambda i: (i, 0, 0)),
            pl.BlockSpec(memory_space=pltpu.SMEM),
        ],
        out_specs=pl.BlockSpec((1, c, hw), lambda i: (i, 0, 0)),
        out_shape=jax.ShapeDtypeStruct((b, c, hw), jnp.float32),
    )(x, mask_id)
    return out.reshape(b, c, h, w)
